# Initial kernel scaffold; baseline (speedup 1.0000x reference)
import functools

import jax
import jax.numpy as jnp
from jax import lax
from jax.experimental import pallas as pl
from jax.experimental.pallas import tpu as pltpu
from jax.experimental.pallas import tpu_sc as plsc

_N = 10000
_E = 320000
_H = 128
_NCORES = 2
_NSUB = 16
_K = 128
_CHUNKS = 157
_EPT = _CHUNKS * _K
_EPAD = _NSUB * _EPT
_RPT = 2 * _N // _NSUB
_WB = 250
_BLK = 400


def _lk(v):
    return jnp.where(v > 0, v, 0.01 * v)


def _proj_body(des_r, tw_r, nc_r, Wd_r, bd_r, Wt_r, bt_r, Wn_r, bn_r,
               Wc_r, bc_r, xL_r, xR_r):
    x = _lk(jnp.dot(des_r[...], Wd_r[...], preferred_element_type=jnp.float32)
            + bd_r[...])
    x = x + _lk(jnp.dot(tw_r[...], Wt_r[...],
                        preferred_element_type=jnp.float32) + bt_r[...])
    nc = nc_r[...]
    x = x + _lk(jnp.dot(nc, Wn_r[...],
                        preferred_element_type=jnp.float32) + bn_r[...])
    x = x + _lk(jnp.dot(nc, Wc_r[...],
                        preferred_element_type=jnp.float32) + bc_r[...])
    xL_r[...] = x[:, :64]
    xR_r[...] = x[:, 64:]


def _layer_body(xL_r, xR_r, a0L_r, a0R_r, a1L_r, a1R_r, c0_r, c1_r,
                Wroot_r, Wr0_r, Wr1_r, b_r, yL_r, yR_r):
    x = jnp.concatenate([xL_r[...], xR_r[...]], axis=1)
    a0 = jnp.concatenate([a0L_r[...], a0R_r[...]], axis=1)
    a1 = jnp.concatenate([a1L_r[...], a1R_r[...]], axis=1)
    c0 = jnp.maximum(c0_r[...][:, 0:1], 1.0)
    c1 = jnp.maximum(c1_r[...][:, 0:1], 1.0)
    out = jnp.dot(x, Wroot_r[...], preferred_element_type=jnp.float32)
    out = out + b_r[...]
    out = out + jnp.dot(a0 / c0, Wr0_r[...],
                        preferred_element_type=jnp.float32)
    out = out + jnp.dot(a1 / c1, Wr1_r[...],
                        preferred_element_type=jnp.float32)
    out = _lk(out)
    yL_r[...] = out[:, :64]
    yR_r[...] = out[:, 64:]


def _head_body(xL_r, xR_r, W1_r, b1_r, W2_r, b2_r, o_r):
    x = jnp.concatenate([xL_r[...], xR_r[...]], axis=1)
    h = jnp.maximum(
        jnp.dot(x, W1_r[...], preferred_element_type=jnp.float32) + b1_r[...],
        0.0)
    o_r[...] = jnp.dot(h, W2_r[...],
                       preferred_element_type=jnp.float32) + b2_r[...]


def _row_spec(cols):
    return pl.BlockSpec((_BLK, cols), lambda i: (i, 0))


def _full_spec(shape):
    return pl.BlockSpec(shape, lambda i: tuple(0 for _ in shape))


def _agg_body(xL, xR, srcp, dstp, typp, ones_c, z64, z16,
              aggL, aggR, cnt_out,
              src_v, dst_v, typ_v, idx_v, rows_v, ones_v, wb_v, wbc_v,
              acc, cnt, sem):
    c = lax.axis_index("c")
    s = lax.axis_index("s")

    pltpu.sync_copy(z64, wb_v)
    pltpu.sync_copy(z16, wbc_v)
    for j in range(_RPT // _WB):
        r0 = s * _RPT + j * _WB
        pltpu.sync_copy(wb_v, acc.at[pl.ds(r0, _WB)])
        pltpu.sync_copy(wbc_v, cnt.at[pl.ds(r0, _WB)])
    pltpu.sync_copy(ones_c, ones_v)
    plsc.subcore_barrier()

    def chunk(i, carry):
        base = s * _EPT + i * _K
        pltpu.sync_copy(srcp.at[pl.ds(base, _K)], src_v)
        pltpu.sync_copy(dstp.at[pl.ds(base, _K)], dst_v)
        pltpu.sync_copy(typp.at[pl.ds(base, _K)], typ_v)
        for j in range(_K // 16):
            d = dst_v[pl.ds(j * 16, 16)]
            t = typ_v[pl.ds(j * 16, 16)]
            idx_v[pl.ds(j * 16, 16)] = t * _N + d

        @pl.when(c == 0)
        def _():
            pltpu.async_copy(xL.at[src_v], rows_v, sem).wait()

        @pl.when(c == 1)
        def _():
            pltpu.async_copy(xR.at[src_v], rows_v, sem).wait()

        pltpu.sync_copy(rows_v, acc.at[idx_v], add=True)
        pltpu.sync_copy(ones_v, cnt.at[idx_v], add=True)
        return carry

    lax.fori_loop(0, _CHUNKS, chunk, 0)
    plsc.subcore_barrier()

    for j in range(_RPT // _WB):
        r0 = s * _RPT + j * _WB

        @pl.when(c == 0)
        def _():
            pltpu.sync_copy(acc.at[pl.ds(r0, _WB)], wb_v)
            pltpu.sync_copy(wb_v, aggL.at[pl.ds(r0, _WB)])
            pltpu.sync_copy(cnt.at[pl.ds(r0, _WB)], wbc_v)
            pltpu.sync_copy(wbc_v, cnt_out.at[pl.ds(r0, _WB)])

        @pl.when(c == 1)
        def _():
            pltpu.sync_copy(acc.at[pl.ds(r0, _WB)], wb_v)
            pltpu.sync_copy(wb_v, aggR.at[pl.ds(r0, _WB)])


def _aggregate(xL, xR, srcp, dstp, typp, ones_c, z64, z16):
    mesh = plsc.VectorSubcoreMesh(core_axis_name="c", subcore_axis_name="s")
    fn = pl.kernel(
        _agg_body,
        mesh=mesh,
        out_type=[
            jax.ShapeDtypeStruct((2 * _N, 64), jnp.float32),
            jax.ShapeDtypeStruct((2 * _N, 64), jnp.float32),
            jax.ShapeDtypeStruct((2 * _N, 16), jnp.float32),
        ],
        scratch_types=[
            pltpu.VMEM((_K,), jnp.int32),
            pltpu.VMEM((_K,), jnp.int32),
            pltpu.VMEM((_K,), jnp.int32),
            pltpu.VMEM((_K,), jnp.int32),
            pltpu.VMEM((_K, 64), jnp.float32),
            pltpu.VMEM((_K, 16), jnp.float32),
            pltpu.VMEM((_WB, 64), jnp.float32),
            pltpu.VMEM((_WB, 16), jnp.float32),
            pltpu.VMEM_SHARED((2 * _N + 8, 64), jnp.float32),
            pltpu.VMEM_SHARED((2 * _N + 8, 16), jnp.float32),
            pltpu.SemaphoreType.DMA,
        ],
    )
    return fn(xL, xR, srcp, dstp, typp, ones_c, z64, z16)


def kernel(des, tweets, num, cat, edge_index, edge_type,
           W_des, b_des, W_tw, b_tw, W_num, b_num, W_cat, b_cat,
           W_root0, W_rel0, b0, W_root1, W_rel1, b1,
           W_m1, b_m1, W_m2, b_m2):
    f32 = jnp.float32
    grid = _N // _BLK

    src = edge_index[0].astype(jnp.int32)
    dst = edge_index[1].astype(jnp.int32)
    typ = edge_type.astype(jnp.int32)
    pad = _EPAD - _E
    srcp = jnp.pad(src, (0, pad))
    dstp = jnp.pad(dst, (0, pad), constant_values=_N)
    typp = jnp.pad(typ, (0, pad), constant_values=1)

    nc = jnp.pad(jnp.concatenate([num, cat], axis=1), ((0, 0), (0, 117)))
    Wn_p = jnp.zeros((128, _H), f32).at[0:5, :].set(W_num)
    Wc_p = jnp.zeros((128, _H), f32).at[5:11, :].set(W_cat)
    W2_p = jnp.zeros((_H, 128), f32).at[:, 0:2].set(W_m2)
    b2_p = jnp.zeros((128,), f32).at[0:2].set(b_m2)

    ones_c = jnp.zeros((_K, 16), f32).at[:, 0].set(1.0)
    z64 = jnp.zeros((_WB, 64), f32)
    z16 = jnp.zeros((_WB, 16), f32)

    row2 = lambda b: b.reshape(1, -1)

    xL, xR = pl.pallas_call(
        _proj_body,
        grid=(grid,),
        in_specs=[_row_spec(768), _row_spec(768), _row_spec(128),
                  _full_spec((768, _H)), _full_spec((1, _H)),
                  _full_spec((768, _H)), _full_spec((1, _H)),
                  _full_spec((128, _H)), _full_spec((1, _H)),
                  _full_spec((128, _H)), _full_spec((1, _H))],
        out_specs=[_row_spec(64), _row_spec(64)],
        out_shape=[jax.ShapeDtypeStruct((_N, 64), f32),
                   jax.ShapeDtypeStruct((_N, 64), f32)],
    )(des, tweets, nc, W_des, row2(b_des), W_tw, row2(b_tw),
      Wn_p, row2(b_num), Wc_p, row2(b_cat))

    def rgcn_layer(xL, xR, W_root, W_rel, b):
        aggL, aggR, cnt = _aggregate(xL, xR, srcp, dstp, typp,
                                     ones_c, z64, z16)
        return pl.pallas_call(
            _layer_body,
            grid=(grid,),
            in_specs=[_row_spec(64)] * 6 + [_row_spec(16)] * 2 +
                     [_full_spec((_H, _H))] * 3 + [_full_spec((1, _H))],
            out_specs=[_row_spec(64), _row_spec(64)],
            out_shape=[jax.ShapeDtypeStruct((_N, 64), f32),
                       jax.ShapeDtypeStruct((_N, 64), f32)],
        )(xL, xR, aggL[:_N], aggR[:_N], aggL[_N:], aggR[_N:],
          cnt[:_N], cnt[_N:], W_root, W_rel[0], W_rel[1], row2(b))

    xL, xR = rgcn_layer(xL, xR, W_root0, W_rel0, b0)
    xL, xR = rgcn_layer(xL, xR, W_root1, W_rel1, b1)

    out = pl.pallas_call(
        _head_body,
        grid=(grid,),
        in_specs=[_row_spec(64), _row_spec(64),
                  _full_spec((_H, _H)), _full_spec((1, _H)),
                  _full_spec((_H, 128)), _full_spec((1, 128))],
        out_specs=_row_spec(128),
        out_shape=jax.ShapeDtypeStruct((_N, 128), f32),
    )(xL, xR, W_m1, row2(b_m1), W2_p, row2(b2_p))

    return out[:, :2]


# speedup vs baseline: 4.7048x; 4.7048x over previous
import functools

import jax
import jax.numpy as jnp
from jax import lax
from jax.experimental import pallas as pl
from jax.experimental.pallas import tpu as pltpu
from jax.experimental.pallas import tpu_sc as plsc

_N = 10000
_E = 320000
_H = 128
_NCORES = 2
_NSUB = 16
_K = 128
_CHUNKS = 157
_EPT = _CHUNKS * _K
_EPAD = _NSUB * _EPT
_NR = 20480
_RPT = _NR // _NSUB
_WB = 256
_BLK = 400


def _lk(v):
    return jnp.where(v > 0, v, 0.01 * v)


def _proj_body(des_r, tw_r, nc_r, Wd_r, bd_r, Wt_r, bt_r, Wn_r, bn_r,
               Wc_r, bc_r, xL_r, xR_r):
    x = _lk(jnp.dot(des_r[...], Wd_r[...], preferred_element_type=jnp.float32)
            + bd_r[...])
    x = x + _lk(jnp.dot(tw_r[...], Wt_r[...],
                        preferred_element_type=jnp.float32) + bt_r[...])
    nc = nc_r[...]
    x = x + _lk(jnp.dot(nc, Wn_r[...],
                        preferred_element_type=jnp.float32) + bn_r[...])
    x = x + _lk(jnp.dot(nc, Wc_r[...],
                        preferred_element_type=jnp.float32) + bc_r[...])
    xL_r[...] = x[:, :64]
    xR_r[...] = x[:, 64:]


def _layer_body(xL_r, xR_r, a0L_r, a0R_r, a1L_r, a1R_r, c0_r, c1_r,
                Wroot_r, Wr0_r, Wr1_r, b_r, yL_r, yR_r):
    x = jnp.concatenate([xL_r[...], xR_r[...]], axis=1)
    a0 = jnp.concatenate([a0L_r[...], a0R_r[...]], axis=1)
    a1 = jnp.concatenate([a1L_r[...], a1R_r[...]], axis=1)
    c0 = jnp.maximum(c0_r[...][:, 0:1], 1.0)
    c1 = jnp.maximum(c1_r[...][:, 0:1], 1.0)
    out = jnp.dot(x, Wroot_r[...], preferred_element_type=jnp.float32)
    out = out + b_r[...]
    out = out + jnp.dot(a0 / c0, Wr0_r[...],
                        preferred_element_type=jnp.float32)
    out = out + jnp.dot(a1 / c1, Wr1_r[...],
                        preferred_element_type=jnp.float32)
    out = _lk(out)
    yL_r[...] = out[:, :64]
    yR_r[...] = out[:, 64:]


def _head_body(xL_r, xR_r, W1_r, b1_r, W2_r, b2_r, o_r):
    x = jnp.concatenate([xL_r[...], xR_r[...]], axis=1)
    h = jnp.maximum(
        jnp.dot(x, W1_r[...], preferred_element_type=jnp.float32) + b1_r[...],
        0.0)
    o_r[...] = jnp.dot(h, W2_r[...],
                       preferred_element_type=jnp.float32) + b2_r[...]


def _row_spec(cols):
    return pl.BlockSpec((_BLK, cols), lambda i: (i, 0))


def _full_spec(shape):
    return pl.BlockSpec(shape, lambda i: tuple(0 for _ in shape))


def _agg_body(xL, xR, srcp, dstp, typp, ones_c, z64, z16,
              aggL, aggR, cnt_out,
              src_v, dst_v, typ_v, idx_v, rows_v, ones_v, wb_v, wbc_v,
              acc, cnt, sem):
    c = lax.axis_index("c")
    s = lax.axis_index("s")

    pltpu.sync_copy(z64, wb_v)
    pltpu.sync_copy(z16, wbc_v)
    for j in range(_RPT // _WB):
        r0 = s * _RPT + j * _WB
        pltpu.sync_copy(wb_v, acc.at[pl.ds(r0, _WB)])
        pltpu.sync_copy(wbc_v, cnt.at[pl.ds(r0, _WB)])
    pltpu.sync_copy(ones_c, ones_v)
    plsc.subcore_barrier()

    def chunk(i, carry):
        base = s * _EPT + i * _K
        pltpu.sync_copy(srcp.at[pl.ds(base, _K)], src_v)
        pltpu.sync_copy(dstp.at[pl.ds(base, _K)], dst_v)
        pltpu.sync_copy(typp.at[pl.ds(base, _K)], typ_v)
        for j in range(_K // 16):
            d = dst_v[pl.ds(j * 16, 16)]
            t = typ_v[pl.ds(j * 16, 16)]
            idx_v[pl.ds(j * 16, 16)] = t * _N + d

        @pl.when(c == 0)
        def _():
            pltpu.async_copy(xL.at[src_v], rows_v, sem).wait()

        @pl.when(c == 1)
        def _():
            pltpu.async_copy(xR.at[src_v], rows_v, sem).wait()

        pltpu.sync_copy(rows_v, acc.at[idx_v], add=True)
        pltpu.sync_copy(ones_v, cnt.at[idx_v], add=True)
        return carry

    lax.fori_loop(0, _CHUNKS, chunk, 0)
    plsc.subcore_barrier()

    for j in range(_RPT // _WB):
        r0 = s * _RPT + j * _WB

        @pl.when(c == 0)
        def _():
            pltpu.sync_copy(acc.at[pl.ds(r0, _WB)], wb_v)
            pltpu.sync_copy(wb_v, aggL.at[pl.ds(r0, _WB)])
            pltpu.sync_copy(cnt.at[pl.ds(r0, _WB)], wbc_v)
            pltpu.sync_copy(wbc_v, cnt_out.at[pl.ds(r0, _WB)])

        @pl.when(c == 1)
        def _():
            pltpu.sync_copy(acc.at[pl.ds(r0, _WB)], wb_v)
            pltpu.sync_copy(wb_v, aggR.at[pl.ds(r0, _WB)])


def _aggregate(xL, xR, srcp, dstp, typp, ones_c, z64, z16):
    mesh = plsc.VectorSubcoreMesh(core_axis_name="c", subcore_axis_name="s")
    fn = pl.kernel(
        _agg_body,
        mesh=mesh,
        out_type=[
            jax.ShapeDtypeStruct((_NR, 64), jnp.float32),
            jax.ShapeDtypeStruct((_NR, 64), jnp.float32),
            jax.ShapeDtypeStruct((_NR, 8), jnp.float32),
        ],
        scratch_types=[
            pltpu.VMEM((_K,), jnp.int32),
            pltpu.VMEM((_K,), jnp.int32),
            pltpu.VMEM((_K,), jnp.int32),
            pltpu.VMEM((_K,), jnp.int32),
            pltpu.VMEM((_K, 64), jnp.float32),
            pltpu.VMEM((_K, 8), jnp.float32),
            pltpu.VMEM((_WB, 64), jnp.float32),
            pltpu.VMEM((_WB, 8), jnp.float32),
            pltpu.VMEM_SHARED((_NR, 64), jnp.float32),
            pltpu.VMEM_SHARED((_NR, 8), jnp.float32),
            pltpu.SemaphoreType.DMA,
        ],
        compiler_params=pltpu.CompilerParams(use_tc_tiling_on_sc=False),
    )
    return fn(xL, xR, srcp, dstp, typp, ones_c, z64, z16)


def kernel(des, tweets, num, cat, edge_index, edge_type,
           W_des, b_des, W_tw, b_tw, W_num, b_num, W_cat, b_cat,
           W_root0, W_rel0, b0, W_root1, W_rel1, b1,
           W_m1, b_m1, W_m2, b_m2):
    f32 = jnp.float32
    grid = _N // _BLK

    src = edge_index[0].astype(jnp.int32)
    dst = edge_index[1].astype(jnp.int32)
    typ = edge_type.astype(jnp.int32)
    pad = _EPAD - _E
    srcp = jnp.pad(src, (0, pad))
    dstp = jnp.pad(dst, (0, pad), constant_values=_N)
    typp = jnp.pad(typ, (0, pad), constant_values=1)

    nc = jnp.pad(jnp.concatenate([num, cat], axis=1), ((0, 0), (0, 117)))
    Wn_p = jnp.zeros((128, _H), f32).at[0:5, :].set(W_num)
    Wc_p = jnp.zeros((128, _H), f32).at[5:11, :].set(W_cat)
    W2_p = jnp.zeros((_H, 128), f32).at[:, 0:2].set(W_m2)
    b2_p = jnp.zeros((128,), f32).at[0:2].set(b_m2)

    ones_c = jnp.zeros((_K, 8), f32).at[:, 0].set(1.0)
    z64 = jnp.zeros((_WB, 64), f32)
    z16 = jnp.zeros((_WB, 8), f32)

    row2 = lambda b: b.reshape(1, -1)

    xL, xR = pl.pallas_call(
        _proj_body,
        grid=(grid,),
        in_specs=[_row_spec(768), _row_spec(768), _row_spec(128),
                  _full_spec((768, _H)), _full_spec((1, _H)),
                  _full_spec((768, _H)), _full_spec((1, _H)),
                  _full_spec((128, _H)), _full_spec((1, _H)),
                  _full_spec((128, _H)), _full_spec((1, _H))],
        out_specs=[_row_spec(64), _row_spec(64)],
        out_shape=[jax.ShapeDtypeStruct((_N, 64), f32),
                   jax.ShapeDtypeStruct((_N, 64), f32)],
    )(des, tweets, nc, W_des, row2(b_des), W_tw, row2(b_tw),
      Wn_p, row2(b_num), Wc_p, row2(b_cat))

    def rgcn_layer(xL, xR, W_root, W_rel, b):
        aggL, aggR, cnt = _aggregate(xL, xR, srcp, dstp, typp,
                                     ones_c, z64, z16)
        return pl.pallas_call(
            _layer_body,
            grid=(grid,),
            in_specs=[_row_spec(64)] * 6 + [_row_spec(8)] * 2 +
                     [_full_spec((_H, _H))] * 3 + [_full_spec((1, _H))],
            out_specs=[_row_spec(64), _row_spec(64)],
            out_shape=[jax.ShapeDtypeStruct((_N, 64), f32),
                       jax.ShapeDtypeStruct((_N, 64), f32)],
        )(xL, xR, aggL[:_N], aggR[:_N], aggL[_N:2 * _N], aggR[_N:2 * _N],
          cnt[:_N], cnt[_N:2 * _N], W_root, W_rel[0], W_rel[1], row2(b))

    xL, xR = rgcn_layer(xL, xR, W_root0, W_rel0, b0)
    xL, xR = rgcn_layer(xL, xR, W_root1, W_rel1, b1)

    out = pl.pallas_call(
        _head_body,
        grid=(grid,),
        in_specs=[_row_spec(64), _row_spec(64),
                  _full_spec((_H, _H)), _full_spec((1, _H)),
                  _full_spec((_H, 128)), _full_spec((1, 128))],
        out_specs=_row_spec(128),
        out_shape=jax.ShapeDtypeStruct((_N, 128), f32),
    )(xL, xR, W_m1, row2(b_m1), W2_p, row2(b2_p))

    return out[:, :2]


# speedup vs baseline: 4.9885x; 1.0603x over previous
import functools

import jax
import jax.numpy as jnp
from jax import lax
from jax.experimental import pallas as pl
from jax.experimental.pallas import tpu as pltpu
from jax.experimental.pallas import tpu_sc as plsc

_N = 10000
_E = 320000
_H = 128
_NCORES = 2
_NSUB = 16
_K = 128
_NBUF = 4
_CHUNKS = 160
_EPT = _CHUNKS * _K
_EPAD = _NSUB * _EPT
_NR = 20480
_RPT = _NR // _NSUB
_WB = 256
_BLK = 400


def _lk(v):
    return jnp.where(v > 0, v, 0.01 * v)


def _proj_body(des_r, tw_r, nc_r, Wd_r, bd_r, Wt_r, bt_r, Wn_r, bn_r,
               Wc_r, bc_r, xL_r, xR_r):
    x = _lk(jnp.dot(des_r[...], Wd_r[...], preferred_element_type=jnp.float32)
            + bd_r[...])
    x = x + _lk(jnp.dot(tw_r[...], Wt_r[...],
                        preferred_element_type=jnp.float32) + bt_r[...])
    nc = nc_r[...]
    x = x + _lk(jnp.dot(nc, Wn_r[...],
                        preferred_element_type=jnp.float32) + bn_r[...])
    x = x + _lk(jnp.dot(nc, Wc_r[...],
                        preferred_element_type=jnp.float32) + bc_r[...])
    xL_r[...] = x[:, :64]
    xR_r[...] = x[:, 64:]


def _layer_body(xL_r, xR_r, a0L_r, a0R_r, a1L_r, a1R_r, c0_r, c1_r,
                Wroot_r, Wr0_r, Wr1_r, b_r, yL_r, yR_r):
    x = jnp.concatenate([xL_r[...], xR_r[...]], axis=1)
    a0 = jnp.concatenate([a0L_r[...], a0R_r[...]], axis=1)
    a1 = jnp.concatenate([a1L_r[...], a1R_r[...]], axis=1)
    c0 = jnp.maximum(c0_r[...][:, 0:1], 1.0)
    c1 = jnp.maximum(c1_r[...][:, 0:1], 1.0)
    out = jnp.dot(x, Wroot_r[...], preferred_element_type=jnp.float32)
    out = out + b_r[...]
    out = out + jnp.dot(a0 / c0, Wr0_r[...],
                        preferred_element_type=jnp.float32)
    out = out + jnp.dot(a1 / c1, Wr1_r[...],
                        preferred_element_type=jnp.float32)
    out = _lk(out)
    yL_r[...] = out[:, :64]
    yR_r[...] = out[:, 64:]


def _head_body(xL_r, xR_r, W1_r, b1_r, W2_r, b2_r, o_r):
    x = jnp.concatenate([xL_r[...], xR_r[...]], axis=1)
    h = jnp.maximum(
        jnp.dot(x, W1_r[...], preferred_element_type=jnp.float32) + b1_r[...],
        0.0)
    o_r[...] = jnp.dot(h, W2_r[...],
                       preferred_element_type=jnp.float32) + b2_r[...]


def _row_spec(cols):
    return pl.BlockSpec((_BLK, cols), lambda i: (i, 0))


def _full_spec(shape):
    return pl.BlockSpec(shape, lambda i: tuple(0 for _ in shape))


def _agg_body(xI, srcp, dstp, typp, ones_c, z64, z16,
              aggL, aggR, cnt_out, *scr):
    nb = _NBUF
    ones_v, cwb = scr[0:2]
    rows = list(scr[2:2 + nb])
    src_v = list(scr[2 + nb:2 + 2 * nb])
    dst_v = list(scr[2 + 2 * nb:2 + 3 * nb])
    typ_v = list(scr[2 + 3 * nb:2 + 4 * nb])
    gidx_v = list(scr[2 + 4 * nb:2 + 5 * nb])
    idx_v = list(scr[2 + 5 * nb:2 + 6 * nb])
    acc, cnt = scr[2 + 6 * nb], scr[3 + 6 * nb]
    esem = list(scr[4 + 6 * nb:4 + 7 * nb])
    gsem = list(scr[4 + 7 * nb:4 + 8 * nb])
    ssem = list(scr[4 + 8 * nb:4 + 9 * nb])

    c = lax.axis_index("c")
    s = lax.axis_index("s")

    pltpu.sync_copy(z64, rows[0])
    pltpu.sync_copy(z16, cwb)
    for j in range(_RPT // _K):
        r0 = s * _RPT + j * _K
        pltpu.sync_copy(rows[0], acc.at[pl.ds(r0, _K)])
        pltpu.sync_copy(cwb, cnt.at[pl.ds(r0, _K)])
    pltpu.sync_copy(ones_c, ones_v)
    plsc.subcore_barrier()

    ebase = s * _EPT
    ngrp = _CHUNKS // nb

    def eissue(cb, b):
        pltpu.async_copy(srcp.at[pl.ds(ebase + cb * _K, _K)], src_v[b],
                         esem[b])
        pltpu.async_copy(dstp.at[pl.ds(ebase + cb * _K, _K)], dst_v[b],
                         esem[b])
        pltpu.async_copy(typp.at[pl.ds(ebase + cb * _K, _K)], typ_v[b],
                         esem[b])

    for b in range(nb):
        eissue(b, b)

    def group(g, carry):
        for b in range(nb):
            cb = g * nb + b

            @pl.when(g > 0)
            def _():
                pltpu.make_async_copy(rows[b], acc.at[idx_v[b]],
                                      ssem[b]).wait()
                pltpu.make_async_copy(ones_v, cnt.at[idx_v[b]],
                                      ssem[b]).wait()

            for r in (src_v[b], dst_v[b], typ_v[b]):
                pltpu.make_async_copy(srcp.at[pl.ds(0, _K)], r,
                                      esem[b]).wait()
            for j in range(_K // 16):
                sl = pl.ds(j * 16, 16)
                gidx_v[b][sl] = src_v[b][sl] * 2 + c
                idx_v[b][sl] = typ_v[b][sl] * _N + dst_v[b][sl]
            pltpu.async_copy(xI.at[gidx_v[b]], rows[b], gsem[b])

            @pl.when(g < ngrp - 1)
            def _():
                eissue(cb + nb, b)

        for b in range(nb):
            pltpu.make_async_copy(xI.at[pl.ds(0, _K)], rows[b],
                                  gsem[b]).wait()
            pltpu.async_copy(rows[b], acc.at[idx_v[b]], ssem[b], add=True)
            pltpu.async_copy(ones_v, cnt.at[idx_v[b]], ssem[b], add=True)
        return carry

    lax.fori_loop(0, ngrp, group, 0)
    for b in range(nb):
        pltpu.make_async_copy(rows[b], acc.at[idx_v[b]], ssem[b]).wait()
        pltpu.make_async_copy(ones_v, cnt.at[idx_v[b]], ssem[b]).wait()
    plsc.subcore_barrier()

    for j in range(_RPT // _K):
        r0 = s * _RPT + j * _K

        @pl.when(c == 0)
        def _():
            pltpu.sync_copy(acc.at[pl.ds(r0, _K)], rows[0])
            pltpu.sync_copy(rows[0], aggL.at[pl.ds(r0, _K)])
            pltpu.sync_copy(cnt.at[pl.ds(r0, _K)], cwb)
            pltpu.sync_copy(cwb, cnt_out.at[pl.ds(r0, _K)])

        @pl.when(c == 1)
        def _():
            pltpu.sync_copy(acc.at[pl.ds(r0, _K)], rows[0])
            pltpu.sync_copy(rows[0], aggR.at[pl.ds(r0, _K)])


def _aggregate(xI, srcp, dstp, typp, ones_c, z64, z16):
    mesh = plsc.VectorSubcoreMesh(core_axis_name="c", subcore_axis_name="s")
    fn = pl.kernel(
        _agg_body,
        mesh=mesh,
        out_type=[
            jax.ShapeDtypeStruct((_NR, 64), jnp.float32),
            jax.ShapeDtypeStruct((_NR, 64), jnp.float32),
            jax.ShapeDtypeStruct((_NR, 8), jnp.float32),
        ],
        scratch_types=(
            [pltpu.VMEM((_K, 8), jnp.float32),
             pltpu.VMEM((_K, 8), jnp.float32)]
            + [pltpu.VMEM((_K, 64), jnp.float32)] * _NBUF
            + [pltpu.VMEM((_K,), jnp.int32)] * (5 * _NBUF)
            + [pltpu.VMEM_SHARED((_NR, 64), jnp.float32),
               pltpu.VMEM_SHARED((_NR, 8), jnp.float32)]
            + [pltpu.SemaphoreType.DMA] * (3 * _NBUF)
        ),
        compiler_params=pltpu.CompilerParams(use_tc_tiling_on_sc=False),
    )
    return fn(xI, srcp, dstp, typp, ones_c, z64, z16)


def kernel(des, tweets, num, cat, edge_index, edge_type,
           W_des, b_des, W_tw, b_tw, W_num, b_num, W_cat, b_cat,
           W_root0, W_rel0, b0, W_root1, W_rel1, b1,
           W_m1, b_m1, W_m2, b_m2):
    f32 = jnp.float32
    grid = _N // _BLK

    src = edge_index[0].astype(jnp.int32)
    dst = edge_index[1].astype(jnp.int32)
    typ = edge_type.astype(jnp.int32)
    pad = _EPAD - _E
    srcp = jnp.pad(src, (0, pad))
    dstp = jnp.pad(dst, (0, pad), constant_values=_N)
    typp = jnp.pad(typ, (0, pad), constant_values=1)

    nc = jnp.pad(jnp.concatenate([num, cat], axis=1), ((0, 0), (0, 117)))
    Wn_p = jnp.zeros((128, _H), f32).at[0:5, :].set(W_num)
    Wc_p = jnp.zeros((128, _H), f32).at[5:11, :].set(W_cat)
    W2_p = jnp.zeros((_H, 128), f32).at[:, 0:2].set(W_m2)
    b2_p = jnp.zeros((128,), f32).at[0:2].set(b_m2)

    ones_c = jnp.zeros((_K, 8), f32).at[:, 0].set(1.0)
    z64 = jnp.zeros((_K, 64), f32)
    z16 = jnp.zeros((_K, 8), f32)

    row2 = lambda b: b.reshape(1, -1)

    xL, xR = pl.pallas_call(
        _proj_body,
        grid=(grid,),
        in_specs=[_row_spec(768), _row_spec(768), _row_spec(128),
                  _full_spec((768, _H)), _full_spec((1, _H)),
                  _full_spec((768, _H)), _full_spec((1, _H)),
                  _full_spec((128, _H)), _full_spec((1, _H)),
                  _full_spec((128, _H)), _full_spec((1, _H))],
        out_specs=[_row_spec(64), _row_spec(64)],
        out_shape=[jax.ShapeDtypeStruct((_N, 64), f32),
                   jax.ShapeDtypeStruct((_N, 64), f32)],
    )(des, tweets, nc, W_des, row2(b_des), W_tw, row2(b_tw),
      Wn_p, row2(b_num), Wc_p, row2(b_cat))

    def rgcn_layer(xL, xR, W_root, W_rel, b):
        xI = jnp.stack([xL, xR], axis=1).reshape(2 * _N, 64)
        aggL, aggR, cnt = _aggregate(xI, srcp, dstp, typp,
                                     ones_c, z64, z16)
        return pl.pallas_call(
            _layer_body,
            grid=(grid,),
            in_specs=[_row_spec(64)] * 6 + [_row_spec(8)] * 2 +
                     [_full_spec((_H, _H))] * 3 + [_full_spec((1, _H))],
            out_specs=[_row_spec(64), _row_spec(64)],
            out_shape=[jax.ShapeDtypeStruct((_N, 64), f32),
                       jax.ShapeDtypeStruct((_N, 64), f32)],
        )(xL, xR, aggL[:_N], aggR[:_N], aggL[_N:2 * _N], aggR[_N:2 * _N],
          cnt[:_N], cnt[_N:2 * _N], W_root, W_rel[0], W_rel[1], row2(b))

    xL, xR = rgcn_layer(xL, xR, W_root0, W_rel0, b0)
    xL, xR = rgcn_layer(xL, xR, W_root1, W_rel1, b1)

    out = pl.pallas_call(
        _head_body,
        grid=(grid,),
        in_specs=[_row_spec(64), _row_spec(64),
                  _full_spec((_H, _H)), _full_spec((1, _H)),
                  _full_spec((_H, 128)), _full_spec((1, 128))],
        out_specs=_row_spec(128),
        out_shape=jax.ShapeDtypeStruct((_N, 128), f32),
    )(xL, xR, W_m1, row2(b_m1), W2_p, row2(b2_p))

    return out[:, :2]


# speedup vs baseline: 5.0487x; 1.0121x over previous
import functools

import jax
import jax.numpy as jnp
from jax import lax
from jax.experimental import pallas as pl
from jax.experimental.pallas import tpu as pltpu
from jax.experimental.pallas import tpu_sc as plsc

_N = 10000
_E = 320000
_H = 128
_NCORES = 2
_NSUB = 16
_K = 128
_NBUF = 4
_CHUNKS = 160
_EPT = _CHUNKS * _K
_EPAD = _NSUB * _EPT
_NR = 20480
_RPT = _NR // _NSUB
_WB = 256
_BLK = 400


def _lk(v):
    return jnp.where(v > 0, v, 0.01 * v)


def _proj_body(des_r, tw_r, nc_r, Wd_r, bd_r, Wt_r, bt_r, Wn_r, bn_r,
               Wc_r, bc_r, xL_r, xR_r):
    x = _lk(jnp.dot(des_r[...], Wd_r[...], preferred_element_type=jnp.float32)
            + bd_r[...])
    x = x + _lk(jnp.dot(tw_r[...], Wt_r[...],
                        preferred_element_type=jnp.float32) + bt_r[...])
    nc = nc_r[...]
    x = x + _lk(jnp.dot(nc, Wn_r[...],
                        preferred_element_type=jnp.float32) + bn_r[...])
    x = x + _lk(jnp.dot(nc, Wc_r[...],
                        preferred_element_type=jnp.float32) + bc_r[...])
    xL_r[...] = x[:, :64]
    xR_r[...] = x[:, 64:]


def _layer_body(xL_r, xR_r, a0L_r, a0R_r, a1L_r, a1R_r, c0_r, c1_r,
                Wroot_r, Wr0_r, Wr1_r, b_r, yL_r, yR_r):
    x = jnp.concatenate([xL_r[...], xR_r[...]], axis=1)
    a0 = jnp.concatenate([a0L_r[...], a0R_r[...]], axis=1)
    a1 = jnp.concatenate([a1L_r[...], a1R_r[...]], axis=1)
    c0 = jnp.maximum(c0_r[...][:, 0:1], 1.0)
    c1 = jnp.maximum(c1_r[...][:, 0:1], 1.0)
    out = jnp.dot(x, Wroot_r[...], preferred_element_type=jnp.float32)
    out = out + b_r[...]
    out = out + jnp.dot(a0 / c0, Wr0_r[...],
                        preferred_element_type=jnp.float32)
    out = out + jnp.dot(a1 / c1, Wr1_r[...],
                        preferred_element_type=jnp.float32)
    out = _lk(out)
    yL_r[...] = out[:, :64]
    yR_r[...] = out[:, 64:]


def _head_body(xL_r, xR_r, W1_r, b1_r, W2_r, b2_r, o_r):
    x = jnp.concatenate([xL_r[...], xR_r[...]], axis=1)
    h = jnp.maximum(
        jnp.dot(x, W1_r[...], preferred_element_type=jnp.float32) + b1_r[...],
        0.0)
    o_r[...] = jnp.dot(h, W2_r[...],
                       preferred_element_type=jnp.float32) + b2_r[...]


def _row_spec(cols):
    return pl.BlockSpec((_BLK, cols), lambda i: (i, 0))


def _full_spec(shape):
    return pl.BlockSpec(shape, lambda i: tuple(0 for _ in shape))


def _agg_body(xI, epk, ones_c, z64, z16,
              aggL, aggR, cnt_out, *scr):
    nb = _NBUF
    ones_v, cwb = scr[0:2]
    rows = list(scr[2:2 + nb])
    epk_v = list(scr[2 + nb:2 + 2 * nb])
    gidx_v = list(scr[2 + 2 * nb:2 + 3 * nb])
    idx_v = list(scr[2 + 3 * nb:2 + 4 * nb])
    acc, cnt = scr[2 + 4 * nb], scr[3 + 4 * nb]
    esem = list(scr[4 + 4 * nb:4 + 5 * nb])
    gsem = list(scr[4 + 5 * nb:4 + 6 * nb])
    ssem = list(scr[4 + 6 * nb:4 + 7 * nb])

    c = lax.axis_index("c")
    s = lax.axis_index("s")

    pltpu.sync_copy(z64, rows[0])
    pltpu.sync_copy(z16, cwb)
    nz = _RPT // _K
    for j in range(nz):
        r0 = s * _RPT + j * _K
        pltpu.async_copy(rows[0], acc.at[pl.ds(r0, _K)], gsem[0])
        pltpu.async_copy(cwb, cnt.at[pl.ds(r0, _K)], gsem[1])
    for j in range(nz):
        pltpu.make_async_copy(rows[0], acc.at[pl.ds(0, _K)], gsem[0]).wait()
        pltpu.make_async_copy(cwb, cnt.at[pl.ds(0, _K)], gsem[1]).wait()
    pltpu.sync_copy(ones_c, ones_v)
    plsc.subcore_barrier()

    ebase = s * _EPT
    ngrp = _CHUNKS // nb

    def eissue(cb, b):
        pltpu.async_copy(epk.at[pl.ds(ebase + cb * _K, _K)], epk_v[b],
                         esem[b])

    for b in range(nb):
        eissue(b, b)

    def group(g, carry):
        for b in range(nb):
            cb = g * nb + b

            @pl.when(g > 0)
            def _():
                pltpu.make_async_copy(rows[b], acc.at[idx_v[b]],
                                      ssem[b]).wait()
                pltpu.make_async_copy(ones_v, cnt.at[idx_v[b]],
                                      ssem[b]).wait()

            pltpu.make_async_copy(epk.at[pl.ds(0, _K)], epk_v[b],
                                  esem[b]).wait()
            for j in range(_K // 16):
                sl = pl.ds(j * 16, 16)
                e = epk_v[b][sl]
                gidx_v[b][sl] = lax.shift_right_logical(e, 15) + c
                idx_v[b][sl] = ((lax.shift_right_logical(e, 1) & 32767)
                                + (e & 1) * _N)
            pltpu.async_copy(xI.at[gidx_v[b]], rows[b], gsem[b])

            @pl.when(g < ngrp - 1)
            def _():
                eissue(cb + nb, b)

        for b in range(nb):
            pltpu.make_async_copy(xI.at[pl.ds(0, _K)], rows[b],
                                  gsem[b]).wait()
            pltpu.async_copy(rows[b], acc.at[idx_v[b]], ssem[b], add=True)
            pltpu.async_copy(ones_v, cnt.at[idx_v[b]], ssem[b], add=True)
        return carry

    lax.fori_loop(0, ngrp, group, 0)
    for b in range(nb):
        pltpu.make_async_copy(rows[b], acc.at[idx_v[b]], ssem[b]).wait()
        pltpu.make_async_copy(ones_v, cnt.at[idx_v[b]], ssem[b]).wait()
    plsc.subcore_barrier()

    nw = _RPT // _K
    out = [aggL, aggR]
    for j in range(min(nb, nw)):
        pltpu.async_copy(acc.at[pl.ds(s * _RPT + j * _K, _K)], rows[j],
                         gsem[j])
    for j in range(nw):
        b = j % nb
        r0 = s * _RPT + j * _K
        pltpu.make_async_copy(acc.at[pl.ds(0, _K)], rows[b], gsem[b]).wait()

        @pl.when(c == 0)
        def _():
            pltpu.async_copy(rows[b], aggL.at[pl.ds(r0, _K)], ssem[b])

        @pl.when(c == 1)
        def _():
            pltpu.async_copy(rows[b], aggR.at[pl.ds(r0, _K)], ssem[b])

        @pl.when(c == 0)
        def _():
            pltpu.sync_copy(cnt.at[pl.ds(r0, _K)], cwb)
            pltpu.sync_copy(cwb, cnt_out.at[pl.ds(r0, _K)])

        if j + nb < nw:
            pltpu.make_async_copy(rows[b], aggL.at[pl.ds(0, _K)],
                                  ssem[b]).wait()
            pltpu.async_copy(acc.at[pl.ds(s * _RPT + (j + nb) * _K, _K)],
                             rows[b], gsem[b])
    for j in range(max(0, nw - nb), nw):
        b = j % nb
        pltpu.make_async_copy(rows[b], aggL.at[pl.ds(0, _K)], ssem[b]).wait()


def _aggregate(xI, epk, ones_c, z64, z16):
    mesh = plsc.VectorSubcoreMesh(core_axis_name="c", subcore_axis_name="s")
    fn = pl.kernel(
        _agg_body,
        mesh=mesh,
        out_type=[
            jax.ShapeDtypeStruct((_NR, 64), jnp.float32),
            jax.ShapeDtypeStruct((_NR, 64), jnp.float32),
            jax.ShapeDtypeStruct((_NR, 8), jnp.float32),
        ],
        scratch_types=(
            [pltpu.VMEM((_K, 8), jnp.float32),
             pltpu.VMEM((_K, 8), jnp.float32)]
            + [pltpu.VMEM((_K, 64), jnp.float32)] * _NBUF
            + [pltpu.VMEM((_K,), jnp.int32)] * (3 * _NBUF)
            + [pltpu.VMEM_SHARED((_NR, 64), jnp.float32),
               pltpu.VMEM_SHARED((_NR, 8), jnp.float32)]
            + [pltpu.SemaphoreType.DMA] * (3 * _NBUF)
        ),
        compiler_params=pltpu.CompilerParams(use_tc_tiling_on_sc=False),
    )
    return fn(xI, epk, ones_c, z64, z16)


def kernel(des, tweets, num, cat, edge_index, edge_type,
           W_des, b_des, W_tw, b_tw, W_num, b_num, W_cat, b_cat,
           W_root0, W_rel0, b0, W_root1, W_rel1, b1,
           W_m1, b_m1, W_m2, b_m2):
    f32 = jnp.float32
    grid = _N // _BLK

    src = edge_index[0].astype(jnp.int32)
    dst = edge_index[1].astype(jnp.int32)
    typ = edge_type.astype(jnp.int32)
    pad = _EPAD - _E
    srcp = jnp.pad(src, (0, pad))
    dstp = jnp.pad(dst, (0, pad), constant_values=_N)
    typp = jnp.pad(typ, (0, pad), constant_values=1)
    epk = (srcp << 16) | (dstp << 1) | typp

    nc = jnp.pad(jnp.concatenate([num, cat], axis=1), ((0, 0), (0, 117)))
    Wn_p = jnp.zeros((128, _H), f32).at[0:5, :].set(W_num)
    Wc_p = jnp.zeros((128, _H), f32).at[5:11, :].set(W_cat)
    W2_p = jnp.zeros((_H, 128), f32).at[:, 0:2].set(W_m2)
    b2_p = jnp.zeros((128,), f32).at[0:2].set(b_m2)

    ones_c = jnp.zeros((_K, 8), f32).at[:, 0].set(1.0)
    z64 = jnp.zeros((_K, 64), f32)
    z16 = jnp.zeros((_K, 8), f32)

    row2 = lambda b: b.reshape(1, -1)

    xL, xR = pl.pallas_call(
        _proj_body,
        grid=(grid,),
        in_specs=[_row_spec(768), _row_spec(768), _row_spec(128),
                  _full_spec((768, _H)), _full_spec((1, _H)),
                  _full_spec((768, _H)), _full_spec((1, _H)),
                  _full_spec((128, _H)), _full_spec((1, _H)),
                  _full_spec((128, _H)), _full_spec((1, _H))],
        out_specs=[_row_spec(64), _row_spec(64)],
        out_shape=[jax.ShapeDtypeStruct((_N, 64), f32),
                   jax.ShapeDtypeStruct((_N, 64), f32)],
    )(des, tweets, nc, W_des, row2(b_des), W_tw, row2(b_tw),
      Wn_p, row2(b_num), Wc_p, row2(b_cat))

    def rgcn_layer(xL, xR, W_root, W_rel, b):
        xI = jnp.stack([xL, xR], axis=1).reshape(2 * _N, 64)
        aggL, aggR, cnt = _aggregate(xI, epk, ones_c, z64, z16)
        return pl.pallas_call(
            _layer_body,
            grid=(grid,),
            in_specs=[_row_spec(64)] * 6 + [_row_spec(8)] * 2 +
                     [_full_spec((_H, _H))] * 3 + [_full_spec((1, _H))],
            out_specs=[_row_spec(64), _row_spec(64)],
            out_shape=[jax.ShapeDtypeStruct((_N, 64), f32),
                       jax.ShapeDtypeStruct((_N, 64), f32)],
        )(xL, xR, aggL[:_N], aggR[:_N], aggL[_N:2 * _N], aggR[_N:2 * _N],
          cnt[:_N], cnt[_N:2 * _N], W_root, W_rel[0], W_rel[1], row2(b))

    xL, xR = rgcn_layer(xL, xR, W_root0, W_rel0, b0)
    xL, xR = rgcn_layer(xL, xR, W_root1, W_rel1, b1)

    out = pl.pallas_call(
        _head_body,
        grid=(grid,),
        in_specs=[_row_spec(64), _row_spec(64),
                  _full_spec((_H, _H)), _full_spec((1, _H)),
                  _full_spec((_H, 128)), _full_spec((1, 128))],
        out_specs=_row_spec(128),
        out_shape=jax.ShapeDtypeStruct((_N, 128), f32),
    )(xL, xR, W_m1, row2(b_m1), W2_p, row2(b2_p))

    return out[:, :2]


# speedup vs baseline: 5.2949x; 1.0488x over previous
import functools

import jax
import jax.numpy as jnp
from jax import lax
from jax.experimental import pallas as pl
from jax.experimental.pallas import tpu as pltpu
from jax.experimental.pallas import tpu_sc as plsc

_N = 10000
_E = 320000
_H = 128
_NCORES = 2
_NSUB = 16
_K = 128
_NBUF = 4
_CHUNKS = 160
_EPT = _CHUNKS * _K
_EPAD = _NSUB * _EPT
_NR = 20480
_RPT = _NR // _NSUB
_WB = 256
_BLK = 400


def _lk(v):
    return jnp.where(v > 0, v, 0.01 * v)


def _proj_body(des_r, tw_r, nc_r, Wd_r, bd_r, Wt_r, bt_r, Wn_r, bn_r,
               Wc_r, bc_r, xL_r, xR_r):
    x = _lk(jnp.dot(des_r[...], Wd_r[...], preferred_element_type=jnp.float32)
            + bd_r[...])
    x = x + _lk(jnp.dot(tw_r[...], Wt_r[...],
                        preferred_element_type=jnp.float32) + bt_r[...])
    nc = nc_r[...]
    x = x + _lk(jnp.dot(nc, Wn_r[...],
                        preferred_element_type=jnp.float32) + bn_r[...])
    x = x + _lk(jnp.dot(nc, Wc_r[...],
                        preferred_element_type=jnp.float32) + bc_r[...])
    xL_r[...] = x[:, :64]
    xR_r[...] = x[:, 64:]


def _layer_body(xL_r, xR_r, a0L_r, a0R_r, a1L_r, a1R_r, c0_r, c1_r,
                Wroot_r, Wr0_r, Wr1_r, b_r, yL_r, yR_r):
    x = jnp.concatenate([xL_r[...], xR_r[...]], axis=1)
    a0 = jnp.concatenate([a0L_r[...], a0R_r[...]], axis=1)
    a1 = jnp.concatenate([a1L_r[...], a1R_r[...]], axis=1)
    c0 = jnp.maximum(c0_r[...][:, 0:1], 1.0)
    c1 = jnp.maximum(c1_r[...][:, 0:1], 1.0)
    out = jnp.dot(x, Wroot_r[...], preferred_element_type=jnp.float32)
    out = out + b_r[...]
    out = out + jnp.dot(a0 / c0, Wr0_r[...],
                        preferred_element_type=jnp.float32)
    out = out + jnp.dot(a1 / c1, Wr1_r[...],
                        preferred_element_type=jnp.float32)
    out = _lk(out)
    yL_r[...] = out[:, :64]
    yR_r[...] = out[:, 64:]


def _head_body(xL_r, xR_r, W1_r, b1_r, W2_r, b2_r, o_r):
    x = jnp.concatenate([xL_r[...], xR_r[...]], axis=1)
    h = jnp.maximum(
        jnp.dot(x, W1_r[...], preferred_element_type=jnp.float32) + b1_r[...],
        0.0)
    o_r[...] = jnp.dot(h, W2_r[...],
                       preferred_element_type=jnp.float32) + b2_r[...]


def _row_spec(cols):
    return pl.BlockSpec((_BLK, cols), lambda i: (i, 0))


def _full_spec(shape):
    return pl.BlockSpec(shape, lambda i: tuple(0 for _ in shape))


def _agg_body(with_cnt, *a):
    nb = _NBUF
    if with_cnt:
        xI, epk, ones_c, z64, z16, aggL, aggR, cnt_out = a[:8]
        scr = a[8:]
        ones_v, cwb = scr[0:2]
        scr = scr[2:]
    else:
        xI, epk, z64, aggL, aggR = a[:5]
        scr = a[5:]
        ones_v = cwb = cnt = cnt_out = None
    rows = list(scr[0:nb])
    epk_v = list(scr[nb:2 * nb])
    gidx_v = list(scr[2 * nb:3 * nb])
    idx_v = list(scr[3 * nb:4 * nb])
    acc = scr[4 * nb]
    base = 4 * nb + 1
    if with_cnt:
        cnt = scr[base]
        base += 1
    esem = list(scr[base:base + nb])
    gsem = list(scr[base + nb:base + 2 * nb])
    ssem = list(scr[base + 2 * nb:base + 3 * nb])

    c = lax.axis_index("c")
    s = lax.axis_index("s")

    pltpu.sync_copy(z64, rows[0])
    nz = _RPT // _K
    if with_cnt:
        pltpu.sync_copy(z16, cwb)
    for j in range(nz):
        r0 = s * _RPT + j * _K
        pltpu.async_copy(rows[0], acc.at[pl.ds(r0, _K)], gsem[0])
        if with_cnt:
            pltpu.async_copy(cwb, cnt.at[pl.ds(r0, _K)], gsem[1])
    for j in range(nz):
        pltpu.make_async_copy(rows[0], acc.at[pl.ds(0, _K)], gsem[0]).wait()
        if with_cnt:
            pltpu.make_async_copy(cwb, cnt.at[pl.ds(0, _K)],
                                  gsem[1]).wait()
    if with_cnt:
        pltpu.sync_copy(ones_c, ones_v)
    plsc.subcore_barrier()

    ebase = s * _EPT
    ngrp = _CHUNKS // nb

    def eissue(cb, b):
        pltpu.async_copy(epk.at[pl.ds(ebase + cb * _K, _K)], epk_v[b],
                         esem[b])

    for b in range(nb):
        eissue(b, b)

    def group(g, carry):
        for b in range(nb):
            cb = g * nb + b

            @pl.when(g > 0)
            def _():
                pltpu.make_async_copy(rows[b], acc.at[idx_v[b]],
                                      ssem[b]).wait()
                if with_cnt:
                    pltpu.make_async_copy(ones_v, cnt.at[idx_v[b]],
                                          ssem[b]).wait()

            pltpu.make_async_copy(epk.at[pl.ds(0, _K)], epk_v[b],
                                  esem[b]).wait()
            for j in range(_K // 16):
                sl = pl.ds(j * 16, 16)
                e = epk_v[b][sl]
                gidx_v[b][sl] = lax.shift_right_logical(e, 15) + c
                idx_v[b][sl] = ((lax.shift_right_logical(e, 1) & 32767)
                                + (e & 1) * _N)
            pltpu.async_copy(xI.at[gidx_v[b]], rows[b], gsem[b])

            @pl.when(g < ngrp - 1)
            def _():
                eissue(cb + nb, b)

        for b in range(nb):
            pltpu.make_async_copy(xI.at[pl.ds(0, _K)], rows[b],
                                  gsem[b]).wait()
            pltpu.async_copy(rows[b], acc.at[idx_v[b]], ssem[b], add=True)
            if with_cnt:
                pltpu.async_copy(ones_v, cnt.at[idx_v[b]], ssem[b],
                                 add=True)
        return carry

    lax.fori_loop(0, ngrp, group, 0)
    for b in range(nb):
        pltpu.make_async_copy(rows[b], acc.at[idx_v[b]], ssem[b]).wait()
        if with_cnt:
            pltpu.make_async_copy(ones_v, cnt.at[idx_v[b]], ssem[b]).wait()
    plsc.subcore_barrier()

    nw = _RPT // _K
    out = [aggL, aggR]
    for j in range(min(nb, nw)):
        pltpu.async_copy(acc.at[pl.ds(s * _RPT + j * _K, _K)], rows[j],
                         gsem[j])
    for j in range(nw):
        b = j % nb
        r0 = s * _RPT + j * _K
        pltpu.make_async_copy(acc.at[pl.ds(0, _K)], rows[b], gsem[b]).wait()

        @pl.when(c == 0)
        def _():
            pltpu.async_copy(rows[b], aggL.at[pl.ds(r0, _K)], ssem[b])

        @pl.when(c == 1)
        def _():
            pltpu.async_copy(rows[b], aggR.at[pl.ds(r0, _K)], ssem[b])

        if with_cnt:
            @pl.when(c == 0)
            def _():
                pltpu.sync_copy(cnt.at[pl.ds(r0, _K)], cwb)
                pltpu.sync_copy(cwb, cnt_out.at[pl.ds(r0, _K)])

        if j + nb < nw:
            pltpu.make_async_copy(rows[b], aggL.at[pl.ds(0, _K)],
                                  ssem[b]).wait()
            pltpu.async_copy(acc.at[pl.ds(s * _RPT + (j + nb) * _K, _K)],
                             rows[b], gsem[b])
    for j in range(max(0, nw - nb), nw):
        b = j % nb
        pltpu.make_async_copy(rows[b], aggL.at[pl.ds(0, _K)], ssem[b]).wait()


def _aggregate(xI, epk, ones_c, z64, z16, with_cnt):
    mesh = plsc.VectorSubcoreMesh(core_axis_name="c", subcore_axis_name="s")
    out_type = [
        jax.ShapeDtypeStruct((_NR, 64), jnp.float32),
        jax.ShapeDtypeStruct((_NR, 64), jnp.float32),
    ]
    scratch = (
        [pltpu.VMEM((_K, 64), jnp.float32)] * _NBUF
        + [pltpu.VMEM((_K,), jnp.int32)] * (3 * _NBUF)
        + [pltpu.VMEM_SHARED((_NR, 64), jnp.float32)]
    )
    if with_cnt:
        out_type = out_type + [jax.ShapeDtypeStruct((_NR, 8), jnp.float32)]
        scratch = ([pltpu.VMEM((_K, 8), jnp.float32),
                    pltpu.VMEM((_K, 8), jnp.float32)]
                   + scratch
                   + [pltpu.VMEM_SHARED((_NR, 8), jnp.float32)])
    fn = pl.kernel(
        functools.partial(_agg_body, with_cnt),
        mesh=mesh,
        out_type=out_type,
        scratch_types=scratch + [pltpu.SemaphoreType.DMA] * (3 * _NBUF),
        compiler_params=pltpu.CompilerParams(use_tc_tiling_on_sc=False),
    )
    if with_cnt:
        return fn(xI, epk, ones_c, z64, z16)
    return fn(xI, epk, z64)


def kernel(des, tweets, num, cat, edge_index, edge_type,
           W_des, b_des, W_tw, b_tw, W_num, b_num, W_cat, b_cat,
           W_root0, W_rel0, b0, W_root1, W_rel1, b1,
           W_m1, b_m1, W_m2, b_m2):
    f32 = jnp.float32
    grid = _N // _BLK

    src = edge_index[0].astype(jnp.int32)
    dst = edge_index[1].astype(jnp.int32)
    typ = edge_type.astype(jnp.int32)
    pad = _EPAD - _E
    srcp = jnp.pad(src, (0, pad))
    dstp = jnp.pad(dst, (0, pad), constant_values=_N)
    typp = jnp.pad(typ, (0, pad), constant_values=1)
    epk = (srcp << 16) | (dstp << 1) | typp

    nc = jnp.pad(jnp.concatenate([num, cat], axis=1), ((0, 0), (0, 117)))
    Wn_p = jnp.zeros((128, _H), f32).at[0:5, :].set(W_num)
    Wc_p = jnp.zeros((128, _H), f32).at[5:11, :].set(W_cat)
    W2_p = jnp.zeros((_H, 128), f32).at[:, 0:2].set(W_m2)
    b2_p = jnp.zeros((128,), f32).at[0:2].set(b_m2)

    ones_c = jnp.zeros((_K, 8), f32).at[:, 0].set(1.0)
    z64 = jnp.zeros((_K, 64), f32)
    z16 = jnp.zeros((_K, 8), f32)

    row2 = lambda b: b.reshape(1, -1)

    xL, xR = pl.pallas_call(
        _proj_body,
        grid=(grid,),
        in_specs=[_row_spec(768), _row_spec(768), _row_spec(128),
                  _full_spec((768, _H)), _full_spec((1, _H)),
                  _full_spec((768, _H)), _full_spec((1, _H)),
                  _full_spec((128, _H)), _full_spec((1, _H)),
                  _full_spec((128, _H)), _full_spec((1, _H))],
        out_specs=[_row_spec(64), _row_spec(64)],
        out_shape=[jax.ShapeDtypeStruct((_N, 64), f32),
                   jax.ShapeDtypeStruct((_N, 64), f32)],
    )(des, tweets, nc, W_des, row2(b_des), W_tw, row2(b_tw),
      Wn_p, row2(b_num), Wc_p, row2(b_cat))

    off = _N // _BLK
    lo_spec = pl.BlockSpec((_BLK, 64), lambda i: (i, 0))
    hi_spec = pl.BlockSpec((_BLK, 64), lambda i: (i + off, 0))
    clo_spec = pl.BlockSpec((_BLK, 8), lambda i: (i, 0))
    chi_spec = pl.BlockSpec((_BLK, 8), lambda i: (i + off, 0))

    def rgcn_layer(xL, xR, W_root, W_rel, b, cnt):
        xI = jnp.stack([xL, xR], axis=1).reshape(2 * _N, 64)
        if cnt is None:
            aggL, aggR, cnt = _aggregate(xI, epk, ones_c, z64, z16, True)
        else:
            aggL, aggR = _aggregate(xI, epk, ones_c, z64, z16, False)
        yL, yR = pl.pallas_call(
            _layer_body,
            grid=(grid,),
            in_specs=[_row_spec(64)] * 2 +
                     [lo_spec, lo_spec, hi_spec, hi_spec,
                      clo_spec, chi_spec] +
                     [_full_spec((_H, _H))] * 3 + [_full_spec((1, _H))],
            out_specs=[_row_spec(64), _row_spec(64)],
            out_shape=[jax.ShapeDtypeStruct((_N, 64), f32),
                       jax.ShapeDtypeStruct((_N, 64), f32)],
        )(xL, xR, aggL, aggR, aggL, aggR,
          cnt, cnt, W_root, W_rel[0], W_rel[1], row2(b))
        return yL, yR, cnt

    xL, xR, cnt = rgcn_layer(xL, xR, W_root0, W_rel0, b0, None)
    xL, xR, _ = rgcn_layer(xL, xR, W_root1, W_rel1, b1, cnt)

    out = pl.pallas_call(
        _head_body,
        grid=(grid,),
        in_specs=[_row_spec(64), _row_spec(64),
                  _full_spec((_H, _H)), _full_spec((1, _H)),
                  _full_spec((_H, 128)), _full_spec((1, 128))],
        out_specs=_row_spec(128),
        out_shape=jax.ShapeDtypeStruct((_N, 128), f32),
    )(xL, xR, W_m1, row2(b_m1), W2_p, row2(b2_p))

    return out[:, :2]


# speedup vs baseline: 6.6298x; 1.2521x over previous
import functools

import jax
import jax.numpy as jnp
from jax import lax
from jax.experimental import pallas as pl
from jax.experimental.pallas import tpu as pltpu
from jax.experimental.pallas import tpu_sc as plsc

_N = 10000
_E = 320000
_H = 128
_NCORES = 2
_NSUB = 16
_K = 128
_NBUF = 4
_CHUNKS = 160
_EPT = _CHUNKS * _K
_EPAD = _NSUB * _EPT
_NR = 20480
_RPT = _NR // _NSUB
_WB = 256
_BLK = 400


def _lk(v):
    return jnp.where(v > 0, v, 0.01 * v)


def _proj_body(des_r, tw_r, nc_r, Wd_r, bd_r, Wt_r, bt_r, Wn_r, bn_r,
               Wc_r, bc_r, xL_r, xR_r):
    x = _lk(jnp.dot(des_r[...], Wd_r[...], preferred_element_type=jnp.float32)
            + bd_r[...])
    x = x + _lk(jnp.dot(tw_r[...], Wt_r[...],
                        preferred_element_type=jnp.float32) + bt_r[...])
    nc = nc_r[...]
    x = x + _lk(jnp.dot(nc, Wn_r[...],
                        preferred_element_type=jnp.float32) + bn_r[...])
    x = x + _lk(jnp.dot(nc, Wc_r[...],
                        preferred_element_type=jnp.float32) + bc_r[...])
    xL_r[...] = x[:, :64]
    xR_r[...] = x[:, 64:]


def _layer_body(xL_r, xR_r, a0L_r, a0R_r, a1L_r, a1R_r, c0_r, c1_r,
                Wroot_r, Wr0_r, Wr1_r, b_r, yL_r, yR_r):
    x = jnp.concatenate([xL_r[...], xR_r[...]], axis=1)
    a0 = jnp.concatenate([a0L_r[...], a0R_r[...]], axis=1)
    a1 = jnp.concatenate([a1L_r[...], a1R_r[...]], axis=1)
    c0 = jnp.maximum(c0_r[...][:, 0:1], 1.0)
    c1 = jnp.maximum(c1_r[...][:, 0:1], 1.0)
    out = jnp.dot(x, Wroot_r[...], preferred_element_type=jnp.float32)
    out = out + b_r[...]
    out = out + jnp.dot(a0 / c0, Wr0_r[...],
                        preferred_element_type=jnp.float32)
    out = out + jnp.dot(a1 / c1, Wr1_r[...],
                        preferred_element_type=jnp.float32)
    out = _lk(out)
    yL_r[...] = out[:, :64]
    yR_r[...] = out[:, 64:]


def _head_body(xL_r, xR_r, W1_r, b1_r, W2_r, b2_r, o_r):
    x = jnp.concatenate([xL_r[...], xR_r[...]], axis=1)
    h = jnp.maximum(
        jnp.dot(x, W1_r[...], preferred_element_type=jnp.float32) + b1_r[...],
        0.0)
    o_r[...] = jnp.dot(h, W2_r[...],
                       preferred_element_type=jnp.float32) + b2_r[...]


def _row_spec(cols):
    return pl.BlockSpec((_BLK, cols), lambda i: (i, 0))


def _full_spec(shape):
    return pl.BlockSpec(shape, lambda i: tuple(0 for _ in shape))


def _agg_body(with_cnt, *a):
    nb = _NBUF
    if with_cnt:
        xL, xR, epk, ones_c, z64, z16, aggL, aggR, cnt_out = a[:9]
        scr = a[9:]
        ones_v, cwb = scr[0:2]
        scr = scr[2:]
    else:
        xL, xR, epk, z64, aggL, aggR = a[:6]
        scr = a[6:]
        ones_v = cwb = cnt = cnt_out = None
    rows = list(scr[0:nb])
    epk_v = list(scr[nb:2 * nb])
    gidx_v = list(scr[2 * nb:3 * nb])
    idx_v = list(scr[3 * nb:4 * nb])
    acc = scr[4 * nb]
    base = 4 * nb + 1
    if with_cnt:
        cnt = scr[base]
        base += 1
    esem = list(scr[base:base + nb])
    gsem = list(scr[base + nb:base + 2 * nb])
    ssem = list(scr[base + 2 * nb:base + 3 * nb])

    c = lax.axis_index("c")
    s = lax.axis_index("s")

    pltpu.sync_copy(z64, rows[0])
    nz = _RPT // _K
    if with_cnt:
        pltpu.sync_copy(z16, cwb)
    for j in range(nz):
        r0 = s * _RPT + j * _K
        pltpu.async_copy(rows[0], acc.at[pl.ds(r0, _K)], gsem[0])
        if with_cnt:
            pltpu.async_copy(cwb, cnt.at[pl.ds(r0, _K)], gsem[1])
    for j in range(nz):
        pltpu.make_async_copy(rows[0], acc.at[pl.ds(0, _K)], gsem[0]).wait()
        if with_cnt:
            pltpu.make_async_copy(cwb, cnt.at[pl.ds(0, _K)],
                                  gsem[1]).wait()
    if with_cnt:
        pltpu.sync_copy(ones_c, ones_v)
    plsc.subcore_barrier()

    ebase = s * _EPT
    ngrp = _CHUNKS // nb

    def eissue(cb, b):
        pltpu.async_copy(epk.at[pl.ds(ebase + cb * _K, _K)], epk_v[b],
                         esem[b])

    for b in range(nb):
        eissue(b, b)

    def group(g, carry):
        for b in range(nb):
            cb = g * nb + b

            @pl.when(g > 0)
            def _():
                pltpu.make_async_copy(rows[b], acc.at[idx_v[b]],
                                      ssem[b]).wait()
                if with_cnt:
                    pltpu.make_async_copy(ones_v, cnt.at[idx_v[b]],
                                          ssem[b]).wait()

            pltpu.make_async_copy(epk.at[pl.ds(0, _K)], epk_v[b],
                                  esem[b]).wait()
            for j in range(_K // 16):
                sl = pl.ds(j * 16, 16)
                e = epk_v[b][sl]
                gidx_v[b][sl] = lax.shift_right_logical(e, 16)
                idx_v[b][sl] = ((lax.shift_right_logical(e, 1) & 32767)
                                + (e & 1) * _N)
            @pl.when(c == 0)
            def _():
                pltpu.async_copy(xL.at[gidx_v[b]], rows[b], gsem[b])

            @pl.when(c == 1)
            def _():
                pltpu.async_copy(xR.at[gidx_v[b]], rows[b], gsem[b])

            @pl.when(g < ngrp - 1)
            def _():
                eissue(cb + nb, b)

        for b in range(nb):
            pltpu.make_async_copy(xL.at[pl.ds(0, _K)], rows[b],
                                  gsem[b]).wait()
            pltpu.async_copy(rows[b], acc.at[idx_v[b]], ssem[b], add=True)
            if with_cnt:
                pltpu.async_copy(ones_v, cnt.at[idx_v[b]], ssem[b],
                                 add=True)
        return carry

    lax.fori_loop(0, ngrp, group, 0)
    for b in range(nb):
        pltpu.make_async_copy(rows[b], acc.at[idx_v[b]], ssem[b]).wait()
        if with_cnt:
            pltpu.make_async_copy(ones_v, cnt.at[idx_v[b]], ssem[b]).wait()
    plsc.subcore_barrier()

    nw = _RPT // _K
    out = [aggL, aggR]
    for j in range(min(nb, nw)):
        pltpu.async_copy(acc.at[pl.ds(s * _RPT + j * _K, _K)], rows[j],
                         gsem[j])
    for j in range(nw):
        b = j % nb
        r0 = s * _RPT + j * _K
        pltpu.make_async_copy(acc.at[pl.ds(0, _K)], rows[b], gsem[b]).wait()

        @pl.when(c == 0)
        def _():
            pltpu.async_copy(rows[b], aggL.at[pl.ds(r0, _K)], ssem[b])

        @pl.when(c == 1)
        def _():
            pltpu.async_copy(rows[b], aggR.at[pl.ds(r0, _K)], ssem[b])

        if with_cnt:
            @pl.when(c == 0)
            def _():
                pltpu.sync_copy(cnt.at[pl.ds(r0, _K)], cwb)
                pltpu.sync_copy(cwb, cnt_out.at[pl.ds(r0, _K)])

        if j + nb < nw:
            pltpu.make_async_copy(rows[b], aggL.at[pl.ds(0, _K)],
                                  ssem[b]).wait()
            pltpu.async_copy(acc.at[pl.ds(s * _RPT + (j + nb) * _K, _K)],
                             rows[b], gsem[b])
    for j in range(max(0, nw - nb), nw):
        b = j % nb
        pltpu.make_async_copy(rows[b], aggL.at[pl.ds(0, _K)], ssem[b]).wait()


def _aggregate(xL, xR, epk, ones_c, z64, z16, with_cnt):
    mesh = plsc.VectorSubcoreMesh(core_axis_name="c", subcore_axis_name="s")
    out_type = [
        jax.ShapeDtypeStruct((_NR, 64), jnp.float32),
        jax.ShapeDtypeStruct((_NR, 64), jnp.float32),
    ]
    scratch = (
        [pltpu.VMEM((_K, 64), jnp.float32)] * _NBUF
        + [pltpu.VMEM((_K,), jnp.int32)] * (3 * _NBUF)
        + [pltpu.VMEM_SHARED((_NR, 64), jnp.float32)]
    )
    if with_cnt:
        out_type = out_type + [jax.ShapeDtypeStruct((_NR, 8), jnp.float32)]
        scratch = ([pltpu.VMEM((_K, 8), jnp.float32),
                    pltpu.VMEM((_K, 8), jnp.float32)]
                   + scratch
                   + [pltpu.VMEM_SHARED((_NR, 8), jnp.float32)])
    fn = pl.kernel(
        functools.partial(_agg_body, with_cnt),
        mesh=mesh,
        out_type=out_type,
        scratch_types=scratch + [pltpu.SemaphoreType.DMA] * (3 * _NBUF),
        compiler_params=pltpu.CompilerParams(use_tc_tiling_on_sc=False),
    )
    if with_cnt:
        return fn(xL, xR, epk, ones_c, z64, z16)
    return fn(xL, xR, epk, z64)


def kernel(des, tweets, num, cat, edge_index, edge_type,
           W_des, b_des, W_tw, b_tw, W_num, b_num, W_cat, b_cat,
           W_root0, W_rel0, b0, W_root1, W_rel1, b1,
           W_m1, b_m1, W_m2, b_m2):
    f32 = jnp.float32
    grid = _N // _BLK

    src = edge_index[0].astype(jnp.int32)
    dst = edge_index[1].astype(jnp.int32)
    typ = edge_type.astype(jnp.int32)
    pad = _EPAD - _E
    srcp = jnp.pad(src, (0, pad))
    dstp = jnp.pad(dst, (0, pad), constant_values=_N)
    typp = jnp.pad(typ, (0, pad), constant_values=1)
    epk = (srcp << 16) | (dstp << 1) | typp

    nc = jnp.pad(jnp.concatenate([num, cat], axis=1), ((0, 0), (0, 117)))
    Wn_p = jnp.zeros((128, _H), f32).at[0:5, :].set(W_num)
    Wc_p = jnp.zeros((128, _H), f32).at[5:11, :].set(W_cat)
    W2_p = jnp.zeros((_H, 128), f32).at[:, 0:2].set(W_m2)
    b2_p = jnp.zeros((128,), f32).at[0:2].set(b_m2)

    ones_c = jnp.zeros((_K, 8), f32).at[:, 0].set(1.0)
    z64 = jnp.zeros((_K, 64), f32)
    z16 = jnp.zeros((_K, 8), f32)

    row2 = lambda b: b.reshape(1, -1)

    xL, xR = pl.pallas_call(
        _proj_body,
        grid=(grid,),
        in_specs=[_row_spec(768), _row_spec(768), _row_spec(128),
                  _full_spec((768, _H)), _full_spec((1, _H)),
                  _full_spec((768, _H)), _full_spec((1, _H)),
                  _full_spec((128, _H)), _full_spec((1, _H)),
                  _full_spec((128, _H)), _full_spec((1, _H))],
        out_specs=[_row_spec(64), _row_spec(64)],
        out_shape=[jax.ShapeDtypeStruct((_N, 64), f32),
                   jax.ShapeDtypeStruct((_N, 64), f32)],
    )(des, tweets, nc, W_des, row2(b_des), W_tw, row2(b_tw),
      Wn_p, row2(b_num), Wc_p, row2(b_cat))

    off = _N // _BLK
    lo_spec = pl.BlockSpec((_BLK, 64), lambda i: (i, 0))
    hi_spec = pl.BlockSpec((_BLK, 64), lambda i: (i + off, 0))
    clo_spec = pl.BlockSpec((_BLK, 8), lambda i: (i, 0))
    chi_spec = pl.BlockSpec((_BLK, 8), lambda i: (i + off, 0))

    def rgcn_layer(xL, xR, W_root, W_rel, b, cnt):
        if cnt is None:
            aggL, aggR, cnt = _aggregate(xL, xR, epk, ones_c, z64, z16,
                                         True)
        else:
            aggL, aggR = _aggregate(xL, xR, epk, ones_c, z64, z16, False)
        yL, yR = pl.pallas_call(
            _layer_body,
            grid=(grid,),
            in_specs=[_row_spec(64)] * 2 +
                     [lo_spec, lo_spec, hi_spec, hi_spec,
                      clo_spec, chi_spec] +
                     [_full_spec((_H, _H))] * 3 + [_full_spec((1, _H))],
            out_specs=[_row_spec(64), _row_spec(64)],
            out_shape=[jax.ShapeDtypeStruct((_N, 64), f32),
                       jax.ShapeDtypeStruct((_N, 64), f32)],
        )(xL, xR, aggL, aggR, aggL, aggR,
          cnt, cnt, W_root, W_rel[0], W_rel[1], row2(b))
        return yL, yR, cnt

    xL, xR, cnt = rgcn_layer(xL, xR, W_root0, W_rel0, b0, None)
    xL, xR, _ = rgcn_layer(xL, xR, W_root1, W_rel1, b1, cnt)

    out = pl.pallas_call(
        _head_body,
        grid=(grid,),
        in_specs=[_row_spec(64), _row_spec(64),
                  _full_spec((_H, _H)), _full_spec((1, _H)),
                  _full_spec((_H, 128)), _full_spec((1, 128))],
        out_specs=_row_spec(128),
        out_shape=jax.ShapeDtypeStruct((_N, 128), f32),
    )(xL, xR, W_m1, row2(b_m1), W2_p, row2(b2_p))

    return out[:, :2]
